# baseline (speedup 1.0000x reference)
import functools

import jax
import jax.numpy as jnp
from jax import lax
from jax.experimental import pallas as pl
from jax.experimental.pallas import tpu as pltpu
from jax.experimental.pallas import tpu_sc as plsc

B = 16384
L = 20
D = 16
LANES = 16
IDX_W = 128


def _allsum(v):
    lane = lax.iota(jnp.int32, LANES)
    dn = lax.GatherDimensionNumbers(
        offset_dims=(), collapsed_slice_dims=(0,), start_index_map=(0,))
    for sh in (8, 4, 2, 1):
        perm = lax.gather(v, (lane ^ sh)[:, None], dn, slice_sizes=(1,),
                          mode=lax.GatherScatterMode.PROMISE_IN_BOUNDS)
        v = v + perm
    return v


def _rsqrt_vec(x):
    i = lax.bitcast_convert_type(x, jnp.int32)
    i = jnp.int32(0x5F3759DF) - lax.shift_right_logical(i, 1)
    y = lax.bitcast_convert_type(i, jnp.float32)
    y = y * (1.5 - 0.5 * x * y * y)
    y = y * (1.5 - 0.5 * x * y * y)
    return y


def _make_encoder(nc, ns):
    nw = nc * ns
    pw = B // nw
    ch = 256
    chunks = pw // ch
    rows_per_chunk = ch * L
    g_per_chunk = rows_per_chunk // IDX_W

    mesh = plsc.VectorSubcoreMesh(core_axis_name="c", subcore_axis_name="s")

    @functools.partial(
        pl.kernel,
        out_type=jax.ShapeDtypeStruct((B, D), jnp.float32),
        mesh=mesh,
        compiler_params=pltpu.CompilerParams(use_tc_tiling_on_sc=False),
        scratch_types=[
            pltpu.VMEM((g_per_chunk, IDX_W), jnp.int32),
            pltpu.VMEM((rows_per_chunk, D), jnp.float32),
            pltpu.VMEM((ch, D), jnp.float32),
            pltpu.SemaphoreType.DMA,
        ],
    )
    def encode(idx_hbm, table_hbm, out_hbm, idx_v, rows_v, out_v, sem):
        wid = lax.axis_index("s") * nc + lax.axis_index("c")
        for c in range(chunks):
            base_pt = pl.multiple_of(wid * pw + c * ch, 8)
            gbase = pl.multiple_of((wid * pw * L + c * rows_per_chunk) // IDX_W, 8)
            pltpu.sync_copy(idx_hbm.at[pl.ds(gbase, g_per_chunk)], idx_v)
            copies = [
                pltpu.async_copy(
                    table_hbm.at[idx_v.at[g]],
                    rows_v.at[pl.ds(g * IDX_W, IDX_W)],
                    sem,
                )
                for g in range(g_per_chunk)
            ]
            for cp in copies:
                cp.wait()

            def point_body(p, carry):
                rbase = p * L
                acc = jnp.zeros((LANES,), jnp.float32)
                for j in range(L):
                    v = rows_v[rbase + j]
                    acc = acc + v * _rsqrt_vec(_allsum(v * v))
                s2 = _allsum(acc * acc)
                out_v[p] = acc * _rsqrt_vec(s2)
                return carry

            lax.fori_loop(0, ch, point_body, 0)
            pltpu.sync_copy(out_v, out_hbm.at[pl.ds(base_pt, ch)])

    return encode


def kernel(indices, table):
    info = plsc.get_sparse_core_info()
    enc = _make_encoder(info.num_cores, info.num_subcores)
    idx = indices.astype(jnp.int32).reshape(B * L // IDX_W, IDX_W)
    return enc(idx, table)


# speedup vs baseline: 1.0110x; 1.0110x over previous
import functools

import jax
import jax.numpy as jnp
from jax import lax
from jax.experimental import pallas as pl
from jax.experimental.pallas import tpu as pltpu
from jax.experimental.pallas import tpu_sc as plsc

B = 16384
L = 20
D = 16
LANES = 16


def _allsum(v):
    lane = lax.iota(jnp.int32, LANES)
    dn = lax.GatherDimensionNumbers(
        offset_dims=(), collapsed_slice_dims=(0,), start_index_map=(0,))
    for sh in (8, 4, 2, 1):
        perm = lax.gather(v, (lane ^ sh)[:, None], dn, slice_sizes=(1,),
                          mode=lax.GatherScatterMode.PROMISE_IN_BOUNDS)
        v = v + perm
    return v


def _rsqrt_vec(x):
    i = lax.bitcast_convert_type(x, jnp.int32)
    i = jnp.int32(0x5F3759DF) - lax.shift_right_logical(i, 1)
    y = lax.bitcast_convert_type(i, jnp.float32)
    y = y * (1.5 - 0.5 * x * y * y)
    y = y * (1.5 - 0.5 * x * y * y)
    return y


def _make_encoder(nc, ns):
    nw = nc * ns
    pw = B // nw
    ch = 128
    chunks = pw // ch

    mesh = plsc.VectorSubcoreMesh(core_axis_name="c", subcore_axis_name="s")

    @functools.partial(
        pl.kernel,
        out_type=jax.ShapeDtypeStruct((B, D), jnp.float32),
        mesh=mesh,
        compiler_params=pltpu.CompilerParams(use_tc_tiling_on_sc=False),
        scratch_types=[
            pltpu.VMEM((L, pw), jnp.int32),
            pltpu.VMEM((2, L, ch, D), jnp.float32),
            pltpu.VMEM((ch, D), jnp.float32),
            pltpu.SemaphoreType.DMA((2,)),
        ],
    )
    def encode(idxt_hbm, table_hbm, out_hbm, idx_v, rows_v, out_v, sem):
        wid = lax.axis_index("s") * nc + lax.axis_index("c")
        base_pt0 = pl.multiple_of(wid * pw, 8)
        for j in range(L):
            pltpu.sync_copy(idxt_hbm.at[j, pl.ds(base_pt0, pw)], idx_v.at[j])

        def issue(c):
            buf = c % 2
            off = pl.multiple_of(c * ch, 8)
            return [
                pltpu.async_copy(
                    table_hbm.at[idx_v.at[j, pl.ds(off, ch)]],
                    rows_v.at[buf, j],
                    sem.at[buf],
                )
                for j in range(L)
            ]

        pending = {0: issue(0)}
        for c in range(chunks):
            if c + 1 < chunks:
                pending[c + 1] = issue(c + 1)
            for cp in pending.pop(c):
                cp.wait()
            buf = c % 2

            def point_body(p, carry, buf=buf):
                acc = jnp.zeros((LANES,), jnp.float32)
                for j in range(L):
                    v = rows_v[buf, j, p]
                    acc = acc + v * _rsqrt_vec(_allsum(v * v))
                s2 = _allsum(acc * acc)
                out_v[p] = acc * _rsqrt_vec(s2)
                return carry

            lax.fori_loop(0, ch, point_body, 0)
            base_pt = pl.multiple_of(wid * pw + c * ch, 8)
            pltpu.sync_copy(out_v, out_hbm.at[pl.ds(base_pt, ch)])

    return encode


def kernel(indices, table):
    info = plsc.get_sparse_core_info()
    enc = _make_encoder(info.num_cores, info.num_subcores)
    return enc(indices.astype(jnp.int32).T, table)
